# baseline (speedup 1.0000x reference)
import functools

import jax
import jax.numpy as jnp
from jax import lax
from jax.experimental import pallas as pl
from jax.experimental.pallas import tpu as pltpu

D_MODEL = 2048
HIDDEN = 2048
N_EXP = 8
T_BLK = 256


def _gate_weights(x_blk, wg):
    logits = lax.dot_general(
        x_blk.astype(jnp.bfloat16), wg.astype(jnp.bfloat16),
        (((1,), (1,)), ((), ())),
        preferred_element_type=jnp.float32)
    idx = lax.broadcasted_iota(jnp.int32, logits.shape, 1)
    m1 = jnp.max(logits, axis=1, keepdims=True)
    a1 = jnp.min(jnp.where(logits == m1, idx, N_EXP), axis=1, keepdims=True)
    masked = jnp.where(idx == a1, -jnp.inf, logits)
    m2 = jnp.max(masked, axis=1, keepdims=True)
    a2 = jnp.min(jnp.where(masked == m2, idx, N_EXP), axis=1, keepdims=True)
    e2 = jnp.exp(m2 - m1)
    w1 = 1.0 / (1.0 + e2)
    w2 = 1.0 - w1
    return jnp.where(idx == a1, w1, 0.0) + jnp.where(idx == a2, w2, 0.0)


def _dense_body(x_ref, wg_ref, w1_ref, b1_ref, w2_ref, b2_ref, o_ref):
    e = pl.program_id(1)
    hb = pl.program_id(2)
    x = x_ref[...]
    gw = _gate_weights(x, wg_ref[...])
    eidx = lax.broadcasted_iota(jnp.int32, gw.shape, 1)
    wi = jnp.sum(jnp.where(eidx == e, gw, 0.0), axis=1, keepdims=True)
    h = lax.dot_general(x, w1_ref[0], (((1,), (1,)), ((), ())),
                        preferred_element_type=jnp.float32)
    h = jnp.maximum(h + b1_ref[0], 0.0)
    oi = lax.dot_general(h, w2_ref[0], (((1,), (1,)), ((), ())),
                         preferred_element_type=jnp.float32)

    @pl.when((e == 0) & (hb == 0))
    def _():
        o_ref[...] = jnp.zeros_like(o_ref)

    @pl.when(hb == 0)
    def _():
        o_ref[...] += wi * b2_ref[0]

    o_ref[...] += wi * oi


def kernel(x, Wg, W1, b1, W2, b2):
    x2 = x.reshape(-1, x.shape[-1])
    T = x2.shape[0]
    H_BLK = HIDDEN // 2
    grid = (T // T_BLK, N_EXP, 2)
    out = pl.pallas_call(
        _dense_body,
        grid=grid,
        in_specs=[
            pl.BlockSpec((T_BLK, D_MODEL), lambda t, e, h: (t, 0)),
            pl.BlockSpec((N_EXP, D_MODEL), lambda t, e, h: (0, 0)),
            pl.BlockSpec((1, H_BLK, D_MODEL), lambda t, e, h: (e, h, 0)),
            pl.BlockSpec((1, 1, H_BLK), lambda t, e, h: (e, 0, h)),
            pl.BlockSpec((1, D_MODEL, H_BLK), lambda t, e, h: (e, 0, h)),
            pl.BlockSpec((1, 1, D_MODEL), lambda t, e, h: (e, 0, 0)),
        ],
        out_specs=pl.BlockSpec((T_BLK, D_MODEL), lambda t, e, h: (t, 0)),
        out_shape=jax.ShapeDtypeStruct((T, D_MODEL), jnp.float32),
        compiler_params=pltpu.CompilerParams(
            dimension_semantics=("arbitrary", "arbitrary", "arbitrary")),
    )(x2, Wg, W1, b1.reshape(N_EXP, 1, HIDDEN), W2,
      b2.reshape(N_EXP, 1, D_MODEL))
    return out.reshape(x.shape)


# speedup vs baseline: 1.8625x; 1.8625x over previous
import functools

import jax
import jax.numpy as jnp
from jax import lax
from jax.experimental import pallas as pl
from jax.experimental.pallas import tpu as pltpu
from jax.experimental.pallas import tpu_sc as plsc

D_MODEL = 2048
HIDDEN = 2048
N_EXP = 8
TOPK = 2
T_TOK = 2048
T_BLK = 256
BT = 256
CAP = 2048
CAP_BLKS = CAP // BT
NROWS = N_EXP * CAP
NB = N_EXP + (TOPK * T_TOK) // BT
H_BLK = HIDDEN // 2

SC_NC = 2
SC_NS = 16
SC_NW = SC_NC * SC_NS


def _route_body(x_ref, wg_ref, pos_ref, w_ref, cnt_ref, cnt_s):
    t = pl.program_id(0)

    @pl.when(t == 0)
    def _():
        cnt_s[...] = jnp.zeros_like(cnt_s)

    logits = lax.dot_general(
        wg_ref[...].astype(jnp.bfloat16), x_ref[...].astype(jnp.bfloat16),
        (((1,), (1,)), ((), ())), preferred_element_type=jnp.float32)
    eidx = lax.broadcasted_iota(jnp.int32, logits.shape, 0)
    m1 = jnp.max(logits, axis=0, keepdims=True)
    a1 = jnp.min(jnp.where(logits == m1, eidx, N_EXP), axis=0, keepdims=True)
    oh0 = (eidx == a1).astype(jnp.float32)
    masked = jnp.where(eidx == a1, -jnp.inf, logits)
    m2 = jnp.max(masked, axis=0, keepdims=True)
    a2 = jnp.min(jnp.where(masked == m2, eidx, N_EXP), axis=0, keepdims=True)
    oh1 = (eidx == a2).astype(jnp.float32)
    e2 = jnp.exp(m2 - m1)
    w1v = 1.0 / (1.0 + e2)
    w2v = 1.0 - w1v

    ti = lax.broadcasted_iota(jnp.int32, (T_BLK, T_BLK), 0)
    tj = lax.broadcasted_iota(jnp.int32, (T_BLK, T_BLK), 1)
    tri = (ti < tj).astype(jnp.float32)
    oh01 = oh0 + oh1
    c01 = lax.dot_general(oh01, tri, (((1,), (0,)), ((), ())),
                          preferred_element_type=jnp.float32)

    lane0 = (lax.broadcasted_iota(jnp.int32, cnt_s.shape, 1) == 0)
    cnt_col = jnp.sum(jnp.where(lane0, cnt_s[...], 0.0), axis=1,
                      keepdims=True)
    r = c01 + cnt_col
    pos0 = jnp.sum(oh0 * r, axis=0, keepdims=True).astype(jnp.int32) + a1 * CAP
    pos1 = jnp.sum(oh1 * r, axis=0, keepdims=True).astype(jnp.int32) + a2 * CAP

    new_cnt = cnt_s[...] + jnp.sum(oh01, axis=1, keepdims=True) * lane0
    cnt_s[...] = new_cnt

    zi = jnp.zeros((N_EXP - 2, T_BLK), jnp.int32)
    pos_ref[...] = jnp.concatenate([pos0, pos1, zi], axis=0)
    zf = jnp.zeros((N_EXP - 2, T_BLK), jnp.float32)
    w_ref[...] = jnp.concatenate([w1v, w2v, zf], axis=0)

    @pl.when(t == pl.num_programs(0) - 1)
    def _():
        cnt_ref[...] = new_cnt.astype(jnp.int32)


def _route(x2, Wg):
    return pl.pallas_call(
        _route_body,
        grid=(T_TOK // T_BLK,),
        in_specs=[
            pl.BlockSpec((T_BLK, D_MODEL), lambda t: (t, 0)),
            pl.BlockSpec((N_EXP, D_MODEL), lambda t: (0, 0)),
        ],
        out_specs=[
            pl.BlockSpec((N_EXP, T_BLK), lambda t: (0, t)),
            pl.BlockSpec((N_EXP, T_BLK), lambda t: (0, t)),
            pl.BlockSpec((N_EXP, 128), lambda t: (0, 0)),
        ],
        out_shape=[
            jax.ShapeDtypeStruct((N_EXP, T_TOK), jnp.int32),
            jax.ShapeDtypeStruct((N_EXP, T_TOK), jnp.float32),
            jax.ShapeDtypeStruct((N_EXP, 128), jnp.int32),
        ],
        scratch_shapes=[pltpu.VMEM((N_EXP, 128), jnp.float32)],
    )(x2, Wg)


_DISP_CH = 16


@functools.lru_cache(maxsize=1)
def _make_dispatch_sc():
    @functools.partial(
        pl.kernel,
        out_type=jax.ShapeDtypeStruct((NROWS, D_MODEL), jnp.float32),
        mesh=plsc.VectorSubcoreMesh(core_axis_name="c", subcore_axis_name="s"),
        scratch_types=[
            pltpu.VMEM((_DISP_CH, D_MODEL), jnp.float32),
            pltpu.VMEM((_DISP_CH,), jnp.int32),
            pltpu.VMEM((_DISP_CH,), jnp.int32),
            pltpu.SemaphoreType.DMA,
            pltpu.SemaphoreType.DMA,
        ],
    )
    def disp(x_hbm, pos_hbm, xg_hbm, xrows, idx0, idx1, sem0, sem1):
        wid = lax.axis_index("s") * SC_NC + lax.axis_index("c")
        per_w = T_TOK // SC_NW
        base = wid * per_w
        for c in range(per_w // _DISP_CH):
            tok0 = base + c * _DISP_CH
            pltpu.sync_copy(pos_hbm.at[0, pl.ds(tok0, _DISP_CH)], idx0)
            pltpu.sync_copy(pos_hbm.at[1, pl.ds(tok0, _DISP_CH)], idx1)
            pltpu.sync_copy(x_hbm.at[pl.ds(tok0, _DISP_CH)], xrows)
            cp0 = pltpu.async_copy(xrows, xg_hbm.at[idx0], sem0)
            cp1 = pltpu.async_copy(xrows, xg_hbm.at[idx1], sem1)
            cp0.wait()
            cp1.wait()

    return disp


def _dispatch_sc(x2, pos):
    return _make_dispatch_sc()(x2, pos)


def _ffn_body(e_ref, r_ref, v_ref, xg_ref, w1_ref, b1_ref, w2_ref, b2_ref,
              y_ref):
    s = pl.program_id(0)
    hb = pl.program_id(1)

    @pl.when(v_ref[s] == 1)
    def _():
        h = lax.dot_general(xg_ref[...], w1_ref[0], (((1,), (1,)), ((), ())),
                            preferred_element_type=jnp.float32)
        h = jnp.maximum(h + b1_ref[0], 0.0)
        yp = lax.dot_general(h, w2_ref[0], (((1,), (1,)), ((), ())),
                             preferred_element_type=jnp.float32)

        @pl.when(hb == 0)
        def _():
            y_ref[...] = yp + b2_ref[0]

        @pl.when(hb == 1)
        def _():
            y_ref[...] += yp


def _ffn(xg, W1, b1, W2, b2, e_s, r_s, v_s):
    grid_spec = pltpu.PrefetchScalarGridSpec(
        num_scalar_prefetch=3,
        grid=(NB, 2),
        in_specs=[
            pl.BlockSpec((BT, D_MODEL), lambda s, h, e, r, v: (r[s], 0)),
            pl.BlockSpec((1, H_BLK, D_MODEL), lambda s, h, e, r, v: (e[s], h, 0)),
            pl.BlockSpec((1, 1, H_BLK), lambda s, h, e, r, v: (e[s], 0, h)),
            pl.BlockSpec((1, D_MODEL, H_BLK), lambda s, h, e, r, v: (e[s], 0, h)),
            pl.BlockSpec((1, 1, D_MODEL), lambda s, h, e, r, v: (e[s], 0, 0)),
        ],
        out_specs=pl.BlockSpec((BT, D_MODEL), lambda s, h, e, r, v: (r[s], 0)),
    )
    return pl.pallas_call(
        _ffn_body,
        grid_spec=grid_spec,
        out_shape=jax.ShapeDtypeStruct((NROWS, D_MODEL), jnp.float32),
        compiler_params=pltpu.CompilerParams(
            dimension_semantics=("arbitrary", "arbitrary")),
    )(e_s, r_s, v_s, xg, W1, b1.reshape(N_EXP, 1, HIDDEN), W2,
      b2.reshape(N_EXP, 1, D_MODEL))


_COMB_CH = 8


@functools.lru_cache(maxsize=1)
def _make_combine_sc():
    @functools.partial(
        pl.kernel,
        out_type=jax.ShapeDtypeStruct((T_TOK, D_MODEL), jnp.float32),
        mesh=plsc.VectorSubcoreMesh(core_axis_name="c", subcore_axis_name="s"),
        scratch_types=[
            pltpu.VMEM((_COMB_CH, D_MODEL), jnp.float32),
            pltpu.VMEM((_COMB_CH, D_MODEL), jnp.float32),
            pltpu.VMEM((_COMB_CH, D_MODEL), jnp.float32),
            pltpu.VMEM((_COMB_CH,), jnp.int32),
            pltpu.VMEM((_COMB_CH,), jnp.int32),
            pltpu.VMEM((16,), jnp.float32),
            pltpu.VMEM((16,), jnp.float32),
            pltpu.SemaphoreType.DMA,
            pltpu.SemaphoreType.DMA,
        ],
    )
    def comb(y_hbm, pos_hbm, w_hbm, out_hbm, y0buf, y1buf, obuf,
             idx0, idx1, wv0, wv1, sem0, sem1):
        wid = lax.axis_index("s") * SC_NC + lax.axis_index("c")
        per_w = T_TOK // SC_NW
        base = wid * per_w

        def chunk(c, carry):
            tok0 = base + c * _COMB_CH
            pltpu.sync_copy(pos_hbm.at[0, pl.ds(tok0, _COMB_CH)], idx0)
            pltpu.sync_copy(pos_hbm.at[1, pl.ds(tok0, _COMB_CH)], idx1)
            pltpu.sync_copy(w_hbm.at[0, pl.ds(tok0, _COMB_CH)],
                            wv0.at[pl.ds(0, _COMB_CH)])
            pltpu.sync_copy(w_hbm.at[1, pl.ds(tok0, _COMB_CH)],
                            wv1.at[pl.ds(0, _COMB_CH)])
            g0 = pltpu.async_copy(y_hbm.at[idx0], y0buf, sem0)
            g1 = pltpu.async_copy(y_hbm.at[idx1], y1buf, sem1)
            g0.wait()
            g1.wait()
            w0all = wv0[...]
            w1all = wv1[...]
            dn = lax.GatherDimensionNumbers(
                offset_dims=(), collapsed_slice_dims=(0,),
                start_index_map=(0,))
            for t in range(_COMB_CH):
                tt = jnp.full((16, 1), t, jnp.int32)
                w0v = lax.gather(w0all, tt, dn, (1,),
                                 mode=lax.GatherScatterMode.PROMISE_IN_BOUNDS)
                w1v = lax.gather(w1all, tt, dn, (1,),
                                 mode=lax.GatherScatterMode.PROMISE_IN_BOUNDS)

                def vec(j, carry2):
                    for u in range(8):
                        sl = pl.ds(j * 128 + u * 16, 16)
                        obuf[t, sl] = w0v * y0buf[t, sl] + w1v * y1buf[t, sl]
                    return carry2

                lax.fori_loop(0, D_MODEL // 128, vec, 0)
            pltpu.sync_copy(obuf, out_hbm.at[pl.ds(tok0, _COMB_CH)])
            return carry

        lax.fori_loop(0, per_w // _COMB_CH, chunk, 0)

    return comb


def _combine_sc(y, pos, w):
    return _make_combine_sc()(y, pos, w)


def _schedule(cnt):
    nb = (cnt + BT - 1) // BT
    start = jnp.cumsum(nb) - nb
    total = jnp.sum(nb)
    s = jnp.arange(NB, dtype=jnp.int32)
    e_raw = jnp.clip(
        jnp.searchsorted(start, s, side="right").astype(jnp.int32) - 1,
        0, N_EXP - 1)
    b_in = s - jnp.take(start, e_raw)
    r_raw = e_raw * CAP_BLKS + b_in
    last = jnp.clip(total - 1, 0, NB - 1)
    valid = s < total
    e_s = jnp.where(valid, e_raw, jnp.take(e_raw, last)).astype(jnp.int32)
    r_s = jnp.where(valid, r_raw, jnp.take(r_raw, last)).astype(jnp.int32)
    v_s = valid.astype(jnp.int32)
    return e_s, r_s, v_s


def kernel(x, Wg, W1, b1, W2, b2):
    x2 = x.reshape(-1, x.shape[-1])
    pos, w, cnt = _route(x2, Wg)
    e_s, r_s, v_s = _schedule(cnt[:, 0])
    xg = _dispatch_sc(x2, pos)
    y = _ffn(xg, W1, b1, W2, b2, e_s, r_s, v_s)
    out = _combine_sc(y, pos, w)
    return out.reshape(x.shape)
